# baseline (speedup 1.0000x reference)
import functools

import jax
import jax.numpy as jnp
from jax import lax
from jax.experimental import pallas as pl
from jax.experimental.pallas import tpu as pltpu
from jax.experimental.pallas import tpu_sc as plsc

N_NODES = 10000
N_EDGES = 320000
T_EDGES = N_EDGES + N_NODES
NC, NS, LANES = 2, 16, 16
NW = NC * NS
B_MAX = 128
SBZ = 8192
T_PAD = ((T_EDGES + SBZ + 7) // 8) * 8
N_PTR = ((N_NODES + 1 + 15) // 16) * 16


def _mm_dual(x, wl, wr, b_prev, out_dtype=jnp.float32):
    n, k = x.shape
    d = wl.shape[1]
    bm = 1000
    with_bias = b_prev is not None

    def body(*refs):
        if with_bias:
            x_ref, wl_ref, wr_ref, b_ref, xl_ref, xr_ref = refs
            a = jnp.maximum(x_ref[...] + b_ref[...], 0.0)
        else:
            x_ref, wl_ref, wr_ref, xl_ref, xr_ref = refs
            a = x_ref[...]
        xl_ref[...] = jnp.dot(
            a, wl_ref[...], preferred_element_type=jnp.float32
        ).astype(out_dtype)
        xr_ref[...] = jnp.dot(
            a, wr_ref[...], preferred_element_type=jnp.float32
        ).astype(out_dtype)

    in_specs = [
        pl.BlockSpec((bm, k), lambda i: (i, 0)),
        pl.BlockSpec((k, d), lambda i: (0, 0)),
        pl.BlockSpec((k, d), lambda i: (0, 0)),
    ]
    args = [x, wl, wr]
    if with_bias:
        in_specs.append(pl.BlockSpec((k,), lambda i: (0,)))
        args.append(b_prev)
    return pl.pallas_call(
        body,
        grid=(n // bm,),
        in_specs=in_specs,
        out_specs=[
            pl.BlockSpec((bm, d), lambda i: (i, 0)),
            pl.BlockSpec((bm, d), lambda i: (i, 0)),
        ],
        out_shape=[
            jax.ShapeDtypeStruct((n, d), out_dtype),
            jax.ShapeDtypeStruct((n, d), out_dtype),
        ],
    )(*args)


def _mm_classifier(h, b_prev, wc, bc):
    n, k = h.shape
    d = wc.shape[1]
    bm = 1000

    def body(h_ref, wc_ref, bp_ref, bc_ref, o_ref):
        a = jnp.maximum(h_ref[...] + bp_ref[...], 0.0)
        o_ref[...] = (
            jnp.dot(a, wc_ref[...], preferred_element_type=jnp.float32)
            + bc_ref[...]
        )

    return pl.pallas_call(
        body,
        grid=(n // bm,),
        in_specs=[
            pl.BlockSpec((bm, k), lambda i: (i, 0)),
            pl.BlockSpec((k, d), lambda i: (0, 0)),
            pl.BlockSpec((k,), lambda i: (0,)),
            pl.BlockSpec((d,), lambda i: (0,)),
        ],
        out_specs=pl.BlockSpec((bm, d), lambda i: (i, 0)),
        out_shape=jax.ShapeDtypeStruct((n, d), jnp.float32),
    )(h, wc, b_prev, bc)


def _read_scalar(vref, idx):
    sp = plsc.load_gather(vref, [jnp.broadcast_to(idx, (LANES,))])
    return jnp.max(sp)


@functools.lru_cache(maxsize=None)
def _make_gat_sc(dout: int):
    bsz = min(B_MAX, 32768 // dout)
    kc = dout // LANES
    nblk_super = SBZ // bsz
    mesh = plsc.VectorSubcoreMesh(
        core_axis_name="c", subcore_axis_name="s", num_cores=NC, num_subcores=NS
    )

    @functools.partial(
        pl.kernel,
        out_type=jax.ShapeDtypeStruct((N_NODES, dout), jnp.float32),
        mesh=mesh,
        compiler_params=pltpu.CompilerParams(needs_layout_passes=False),
        scratch_types=[
            pltpu.VMEM((SBZ,), jnp.int32),
            pltpu.VMEM((N_PTR,), jnp.int32),
            pltpu.VMEM((bsz, dout), jnp.float32),
            pltpu.VMEM((bsz, dout), jnp.float32),
            pltpu.VMEM((dout,), jnp.float32),
            pltpu.VMEM((dout,), jnp.float32),
            pltpu.VMEM((dout,), jnp.float32),
            pltpu.VMEM((dout,), jnp.float32),
            pltpu.SemaphoreType.DMA,
            pltpu.SemaphoreType.DMA,
        ],
    )
    def gat(xl_hbm, xr_hbm, src_hbm, rowptr_hbm, att_hbm, out_hbm,
            sidx_v, rowptr_v, arows0, arows1, xr_v, att_v, acc_v, stage_v,
            sem0, sem1):
        wid = lax.axis_index("s") * NC + lax.axis_index("c")
        pltpu.sync_copy(rowptr_hbm, rowptr_v)
        pltpu.sync_copy(att_hbm, att_v)
        n0 = wid * N_NODES // NW
        n1 = (wid + 1) * N_NODES // NW
        e0 = _read_scalar(rowptr_v, n0)
        e1 = _read_scalar(rowptr_v, n1)
        a0 = (e0 // 8) * 8
        n_super = (e1 - a0 + SBZ - 1) // SBZ
        zero16 = jnp.zeros((LANES,), jnp.float32)
        arows = (arows0, arows1)
        sems = (sem0, sem1)

        @plsc.parallel_loop(0, kc, unroll=4)
        def _(k):
            acc_v[pl.ds(k * LANES, LANES)] = zero16

        def flush(n_s, d_vec):
            inv = 1.0 / (d_vec + 1e-16)

            @plsc.parallel_loop(0, kc, unroll=4)
            def _(k):
                sl = pl.ds(k * LANES, LANES)
                stage_v[sl] = acc_v[sl] * inv
                acc_v[sl] = zero16

            pltpu.sync_copy(stage_v, out_hbm.at[n_s])

        pltpu.sync_copy(xr_hbm.at[n0], xr_v)

        def super_body(g, carry):
            sb = a0 + g * SBZ
            pltpu.sync_copy(src_hbm.at[pl.ds(sb, SBZ)], sidx_v)
            nb = jnp.minimum((e1 - sb + bsz - 1) // bsz, nblk_super)
            for s in range(2):
                @pl.when(s < nb)
                def _():
                    pltpu.async_copy(
                        xl_hbm.at[sidx_v.at[pl.ds(s * bsz, bsz)]],
                        arows[s], sems[s],
                    )

            def make_block_body(j, ar):
                bs_g = sb + j * bsz

                def edge_body(e, d_vec):
                    i = e - bs_g

                    @plsc.parallel_loop(
                        0, kc // 2, carry=(zero16, zero16), unroll=4
                    )
                    def score_acc(k, ss):
                        s0, s1 = ss
                        sl0 = pl.ds((2 * k) * LANES, LANES)
                        sl1 = pl.ds((2 * k + 1) * LANES, LANES)
                        t0 = ar[i, sl0] + xr_v[sl0]
                        t1 = ar[i, sl1] + xr_v[sl1]
                        lr0 = jnp.maximum(t0, 0.0) + 0.2 * jnp.minimum(t0, 0.0)
                        lr1 = jnp.maximum(t1, 0.0) + 0.2 * jnp.minimum(t1, 0.0)
                        return (s0 + att_v[sl0] * lr0, s1 + att_v[sl1] * lr1)

                    s0, s1 = score_acc
                    s = jnp.sum(s0 + s1)
                    w_vec = jnp.exp(jnp.broadcast_to(s, (LANES,)))
                    d_vec = d_vec + w_vec

                    @plsc.parallel_loop(0, kc, unroll=8)
                    def _(k):
                        sl = pl.ds(k * LANES, LANES)
                        acc_v[sl] = acc_v[sl] + w_vec * ar[i, sl]

                    return d_vec

                def block_fn(carry2):
                    n, seg_end, d_vec = carry2
                    blk_start = jnp.maximum(e0, bs_g)
                    blk_end = jnp.minimum(e1, bs_g + bsz)

                    def seg_cond(st):
                        return st[0] < blk_end

                    def seg_body(st):
                        pos, n, seg_end, d_vec = st
                        stop = jnp.minimum(seg_end, blk_end)
                        d_vec = lax.fori_loop(pos, stop, edge_body, d_vec)

                        def fin(args):
                            n, _, d_vec = args
                            flush(n, d_vec)
                            n2 = n + 1
                            seg_end2 = _read_scalar(rowptr_v, n2 + 1)

                            @pl.when(n2 < n1)
                            def _():
                                pltpu.sync_copy(xr_hbm.at[n2], xr_v)

                            return (n2, seg_end2, zero16)

                        n, seg_end, d_vec = lax.cond(
                            stop == seg_end, fin, lambda a: a,
                            (n, seg_end, d_vec),
                        )
                        return (stop, n, seg_end, d_vec)

                    _, n, seg_end, d_vec = lax.while_loop(
                        seg_cond, seg_body, (blk_start, n, seg_end, d_vec)
                    )
                    return (n, seg_end, d_vec)

                return block_fn

            def pair_body(gp, carry2):
                for s in range(2):
                    j = gp * 2 + s

                    def do_block(cr, j=j, s=s):
                        pltpu.make_async_copy(
                            xl_hbm.at[sidx_v.at[pl.ds(0, bsz)]],
                            arows[s], sems[s],
                        ).wait()
                        cr = make_block_body(j, arows[s])(cr)

                        @pl.when(j + 2 < nb)
                        def _():
                            pltpu.async_copy(
                                xl_hbm.at[sidx_v.at[pl.ds((j + 2) * bsz, bsz)]],
                                arows[s], sems[s],
                            )

                        return cr

                    carry2 = lax.cond(j < nb, do_block, lambda cr: cr, carry2)
                return carry2

            return lax.fori_loop(0, (nb + 1) // 2, pair_body, carry)

        seg_end0 = _read_scalar(rowptr_v, n0 + 1)
        init = (n0, seg_end0, zero16)
        lax.fori_loop(0, n_super, super_body, init)

    return gat


def kernel(x, edge_index, Wl1, Wr1, att1, b1, Wl2, Wr2, att2, b2,
           Wl3, Wr3, att3, b3, Wc, bc):
    src = edge_index[0].astype(jnp.int32)
    dst = edge_index[1].astype(jnp.int32)
    loop = jnp.arange(N_NODES, dtype=jnp.int32)
    src_full = jnp.concatenate([src, loop])
    dst_full = jnp.concatenate([dst, loop])
    order = jnp.argsort(dst_full)
    src_sorted = src_full[order]
    dst_sorted = dst_full[order]

    rowptr = jnp.searchsorted(
        dst_sorted, jnp.arange(N_NODES + 1, dtype=jnp.int32)
    ).astype(jnp.int32)
    rowptr_p = jnp.concatenate(
        [rowptr, jnp.full((N_PTR - N_NODES - 1,), T_EDGES, dtype=jnp.int32)]
    )

    pad = T_PAD - T_EDGES
    src_p = jnp.concatenate(
        [src_sorted, jnp.zeros((pad,), dtype=jnp.int32)]
    )

    gat1 = _make_gat_sc(1024)
    gat2 = _make_gat_sc(512)
    gat3 = _make_gat_sc(128)

    xl, xr = _mm_dual(x, Wl1, Wr1, None)
    h = gat1(xl, xr, src_p, rowptr_p, att1)
    xl, xr = _mm_dual(h, Wl2, Wr2, b1)
    h = gat2(xl, xr, src_p, rowptr_p, att2)
    xl, xr = _mm_dual(h, Wl3, Wr3, b2)
    h = gat3(xl, xr, src_p, rowptr_p, att3)

    wc_pad = jnp.pad(Wc, ((0, 0), (0, 128 - Wc.shape[1])))
    bc_pad = jnp.pad(bc, (0, 128 - bc.shape[0]))
    out = _mm_classifier(h, b3, wc_pad, bc_pad)
    return out[:, : Wc.shape[1]]


# speedup vs baseline: 1.7655x; 1.7655x over previous
import functools

import jax
import jax.numpy as jnp
from jax import lax
from jax.experimental import pallas as pl
from jax.experimental.pallas import tpu as pltpu
from jax.experimental.pallas import tpu_sc as plsc

N_NODES = 10000
N_EDGES = 320000
T_EDGES = N_EDGES + N_NODES
NC, NS, LANES = 2, 16, 16
NW = NC * NS
B_MAX = 128
SBZ = 8192
T_PAD = ((T_EDGES + SBZ + 7) // 8) * 8
N_PTR = ((N_NODES + 1 + 15) // 16) * 16


def _mm_dual(x, wl, wr, b_prev, out_dtype=jnp.float32):
    n, k = x.shape
    d = wl.shape[1]
    bm = 1000
    with_bias = b_prev is not None

    def body(*refs):
        if with_bias:
            x_ref, wl_ref, wr_ref, b_ref, xl_ref, xr_ref = refs
            a = jnp.maximum(x_ref[...] + b_ref[...], 0.0)
        else:
            x_ref, wl_ref, wr_ref, xl_ref, xr_ref = refs
            a = x_ref[...]
        xl_ref[...] = jnp.dot(
            a, wl_ref[...], preferred_element_type=jnp.float32
        ).astype(out_dtype)
        xr_ref[...] = jnp.dot(
            a, wr_ref[...], preferred_element_type=jnp.float32
        ).astype(out_dtype)

    in_specs = [
        pl.BlockSpec((bm, k), lambda i: (i, 0)),
        pl.BlockSpec((k, d), lambda i: (0, 0)),
        pl.BlockSpec((k, d), lambda i: (0, 0)),
    ]
    args = [x, wl, wr]
    if with_bias:
        in_specs.append(pl.BlockSpec((k,), lambda i: (0,)))
        args.append(b_prev)
    return pl.pallas_call(
        body,
        grid=(n // bm,),
        in_specs=in_specs,
        out_specs=[
            pl.BlockSpec((bm, d), lambda i: (i, 0)),
            pl.BlockSpec((bm, d), lambda i: (i, 0)),
        ],
        out_shape=[
            jax.ShapeDtypeStruct((n, d), out_dtype),
            jax.ShapeDtypeStruct((n, d), out_dtype),
        ],
    )(*args)


def _mm_classifier(h, b_prev, wc, bc):
    n, k = h.shape
    d = wc.shape[1]
    bm = 1000

    def body(h_ref, wc_ref, bp_ref, bc_ref, o_ref):
        a = jnp.maximum(h_ref[...] + bp_ref[...], 0.0)
        o_ref[...] = (
            jnp.dot(a, wc_ref[...], preferred_element_type=jnp.float32)
            + bc_ref[...]
        )

    return pl.pallas_call(
        body,
        grid=(n // bm,),
        in_specs=[
            pl.BlockSpec((bm, k), lambda i: (i, 0)),
            pl.BlockSpec((k, d), lambda i: (0, 0)),
            pl.BlockSpec((k,), lambda i: (0,)),
            pl.BlockSpec((d,), lambda i: (0,)),
        ],
        out_specs=pl.BlockSpec((bm, d), lambda i: (i, 0)),
        out_shape=jax.ShapeDtypeStruct((n, d), jnp.float32),
    )(h, wc, b_prev, bc)


def _read_scalar(vref, idx):
    sp = plsc.load_gather(vref, [jnp.broadcast_to(idx, (LANES,))])
    return jnp.max(sp)


@functools.lru_cache(maxsize=None)
def _make_gat_sc(dout: int):
    bsz = min(B_MAX, 32768 // dout)
    kc = dout // LANES
    nblk_super = SBZ // bsz
    mesh = plsc.VectorSubcoreMesh(
        core_axis_name="c", subcore_axis_name="s", num_cores=NC, num_subcores=NS
    )

    @functools.partial(
        pl.kernel,
        out_type=jax.ShapeDtypeStruct((N_NODES, dout), jnp.float32),
        mesh=mesh,
        compiler_params=pltpu.CompilerParams(needs_layout_passes=False),
        scratch_types=[
            pltpu.VMEM((SBZ,), jnp.int32),
            pltpu.VMEM((SBZ,), jnp.int32),
            pltpu.VMEM((bsz, dout), jnp.float32),
            pltpu.VMEM((bsz, dout), jnp.float32),
            pltpu.VMEM((dout,), jnp.float32),
            pltpu.VMEM((dout,), jnp.float32),
            pltpu.VMEM((dout,), jnp.float32),
            pltpu.VMEM((dout,), jnp.float32),
            pltpu.VMEM((NW,), jnp.int32),
            pltpu.VMEM((NW,), jnp.int32),
            pltpu.SemaphoreType.DMA,
            pltpu.SemaphoreType.DMA,
        ],
    )
    def gat(xl_hbm, xr_hbm, src_hbm, dst_hbm, att_hbm, lo_hbm, hi_hbm,
            out_hbm, sidx_v, didx_v, arows0, arows1, xr_v, att_v, acc_v,
            stage_v, lo_v, hi_v, sem0, sem1):
        wid = lax.axis_index("s") * NC + lax.axis_index("c")
        pltpu.sync_copy(lo_hbm, lo_v)
        pltpu.sync_copy(hi_hbm, hi_v)
        pltpu.sync_copy(att_hbm, att_v)
        e0 = _read_scalar(lo_v, wid)
        e1 = _read_scalar(hi_v, wid)
        a0 = (e0 // 8) * 8
        n_super = (e1 - a0 + SBZ - 1) // SBZ
        zero16 = jnp.zeros((LANES,), jnp.float32)
        arows = (arows0, arows1)
        sems = (sem0, sem1)

        @plsc.parallel_loop(0, kc, unroll=4)
        def _(k):
            acc_v[pl.ds(k * LANES, LANES)] = zero16

        def flush(c_vec, d_vec):
            c_s = jnp.max(c_vec)
            inv = 1.0 / (d_vec + 1e-16)

            @plsc.parallel_loop(0, kc, unroll=4)
            def _(k):
                sl = pl.ds(k * LANES, LANES)
                stage_v[sl] = acc_v[sl] * inv
                acc_v[sl] = zero16

            pltpu.sync_copy(stage_v, out_hbm.at[c_s])

        def super_body(g, carry):
            sb = a0 + g * SBZ
            pltpu.sync_copy(src_hbm.at[pl.ds(sb, SBZ)], sidx_v)
            pltpu.sync_copy(dst_hbm.at[pl.ds(sb, SBZ)], didx_v)
            nb = jnp.minimum((e1 - sb + bsz - 1) // bsz, nblk_super)
            for s in range(2):
                @pl.when(s < nb)
                def _():
                    pltpu.async_copy(
                        xl_hbm.at[sidx_v.at[pl.ds(s * bsz, bsz)]],
                        arows[s], sems[s],
                    )

            def make_edge_body(j, ar):
                def edge_body(i, carry2):
                    c_vec, d_vec = carry2
                    le = j * bsz + i
                    active = (sb + le >= e0) & (sb + le < e1)
                    act_vec = jnp.broadcast_to(active, (LANES,))
                    di_vec = plsc.load_gather(
                        didx_v, [jnp.broadcast_to(le, (LANES,))]
                    )
                    di_vec = jnp.where(act_vec, di_vec, c_vec)
                    changed = jnp.any(di_vec != c_vec)

                    def on_change(cd):
                        c_vec, d_vec = cd

                        @pl.when(jnp.max(c_vec) >= 0)
                        def _():
                            flush(c_vec, d_vec)

                        pltpu.sync_copy(xr_hbm.at[jnp.max(di_vec)], xr_v)
                        return (di_vec, zero16)

                    c_vec, d_vec = lax.cond(
                        changed, on_change, lambda cd: cd, (c_vec, d_vec)
                    )

                    @plsc.parallel_loop(
                        0, kc // 2, carry=(zero16, zero16), unroll=4
                    )
                    def score_acc(k, ss):
                        s0, s1 = ss
                        sl0 = pl.ds((2 * k) * LANES, LANES)
                        sl1 = pl.ds((2 * k + 1) * LANES, LANES)
                        t0 = ar[i, sl0] + xr_v[sl0]
                        t1 = ar[i, sl1] + xr_v[sl1]
                        lr0 = jnp.maximum(t0, 0.0) + 0.2 * jnp.minimum(t0, 0.0)
                        lr1 = jnp.maximum(t1, 0.0) + 0.2 * jnp.minimum(t1, 0.0)
                        return (s0 + att_v[sl0] * lr0, s1 + att_v[sl1] * lr1)

                    s0, s1 = score_acc
                    s = jnp.sum(s0 + s1)
                    w_vec = jnp.exp(jnp.broadcast_to(s, (LANES,)))
                    w_vec = jnp.where(act_vec, w_vec, zero16)
                    d_vec = d_vec + w_vec

                    @plsc.parallel_loop(0, kc, unroll=8)
                    def _(k):
                        sl = pl.ds(k * LANES, LANES)
                        acc_v[sl] = acc_v[sl] + w_vec * ar[i, sl]

                    return (c_vec, d_vec)

                return edge_body

            def pair_body(gp, carry2):
                for s in range(2):
                    j = gp * 2 + s

                    def do_block(cr, j=j, s=s):
                        pltpu.make_async_copy(
                            xl_hbm.at[sidx_v.at[pl.ds(0, bsz)]],
                            arows[s], sems[s],
                        ).wait()
                        cr = lax.fori_loop(
                            0, bsz, make_edge_body(j, arows[s]), cr
                        )

                        @pl.when(j + 2 < nb)
                        def _():
                            pltpu.async_copy(
                                xl_hbm.at[sidx_v.at[pl.ds((j + 2) * bsz, bsz)]],
                                arows[s], sems[s],
                            )

                        return cr

                    carry2 = lax.cond(j < nb, do_block, lambda cr: cr, carry2)
                return carry2

            return lax.fori_loop(0, (nb + 1) // 2, pair_body, carry)

        init = (jnp.full((LANES,), -1, jnp.int32), zero16)
        c_vec, d_vec = lax.fori_loop(0, n_super, super_body, init)

        @pl.when(jnp.max(c_vec) >= 0)
        def _():
            flush(c_vec, d_vec)

    return gat


def kernel(x, edge_index, Wl1, Wr1, att1, b1, Wl2, Wr2, att2, b2,
           Wl3, Wr3, att3, b3, Wc, bc):
    src = edge_index[0].astype(jnp.int32)
    dst = edge_index[1].astype(jnp.int32)
    loop = jnp.arange(N_NODES, dtype=jnp.int32)
    src_full = jnp.concatenate([src, loop])
    dst_full = jnp.concatenate([dst, loop])
    order = jnp.argsort(dst_full)
    src_sorted = src_full[order]
    dst_sorted = dst_full[order]

    splits = jnp.array([w * N_NODES // NW for w in range(NW)], dtype=jnp.int32)
    lo = jnp.searchsorted(dst_sorted, splits).astype(jnp.int32)
    hi = jnp.concatenate(
        [lo[1:], jnp.array([T_EDGES], dtype=jnp.int32)]
    )

    pad = T_PAD - T_EDGES
    zpad = jnp.zeros((pad,), dtype=jnp.int32)
    src_p = jnp.concatenate([src_sorted, zpad])
    dst_p = jnp.concatenate([dst_sorted, zpad])

    gat1 = _make_gat_sc(1024)
    gat2 = _make_gat_sc(512)
    gat3 = _make_gat_sc(128)

    xl, xr = _mm_dual(x, Wl1, Wr1, None)
    h = gat1(xl, xr, src_p, dst_p, att1, lo, hi)
    xl, xr = _mm_dual(h, Wl2, Wr2, b1)
    h = gat2(xl, xr, src_p, dst_p, att2, lo, hi)
    xl, xr = _mm_dual(h, Wl3, Wr3, b2)
    h = gat3(xl, xr, src_p, dst_p, att3, lo, hi)

    wc_pad = jnp.pad(Wc, ((0, 0), (0, 128 - Wc.shape[1])))
    bc_pad = jnp.pad(bc, (0, 128 - bc.shape[0]))
    out = _mm_classifier(h, b3, wc_pad, bc_pad)
    return out[:, : Wc.shape[1]]
